# baseline (speedup 1.0000x reference)
import functools

import jax
import jax.numpy as jnp
from jax import lax
from jax.experimental import pallas as pl
from jax.experimental.pallas import tpu as pltpu
from jax.experimental.pallas import tpu_sc as plsc

N_NODES = 10000
D = 128
N_EDGES = 320000

NC = 2
NS = 16
NW = NC * NS
CHUNK = 128
EPT = ((N_EDGES + NW * CHUNK - 1) // (NW * CHUNK)) * CHUNK
E_PAD = EPT * NW
NCHUNK = EPT // CHUNK
ACC_ROWS = 10240
TRASH = N_NODES
DUMP_ROWS = ACC_ROWS // NS

_mesh = plsc.VectorSubcoreMesh(core_axis_name="c", subcore_axis_name="s")


def _zero_rows(ref, nrows, ncolgrp):
    z = jnp.zeros((16,), jnp.float32)

    def rowfn(i, _):
        for j in range(ncolgrp):
            ref[i, pl.ds(j * 16, 16)] = z
        return 0

    lax.fori_loop(0, nrows, rowfn, 0)


def _unpack(packed2, ch, gi):
    p = packed2[ch, pl.ds(gi * 16, 16)]
    r = jnp.bitwise_and(p, 0xFFFF)
    cc = lax.shift_right_logical(p, 16)
    return r, cc


@functools.partial(
    pl.kernel,
    out_type=jax.ShapeDtypeStruct((NC, ACC_ROWS, D), jnp.float32),
    mesh=_mesh,
    scratch_types=[
        pltpu.VMEM((NCHUNK, CHUNK), jnp.int32),
        pltpu.VMEM((CHUNK,), jnp.int32),
        pltpu.VMEM((CHUNK,), jnp.int32),
        pltpu.VMEM((CHUNK, D), jnp.float32),
        pltpu.VMEM_SHARED((ACC_ROWS, D), jnp.float32),
        pltpu.SemaphoreType.DMA,
        pltpu.SemaphoreType.DMA,
    ],
)
def _deg_call(packed_hbm, out_hbm, packed2, rb0, rb1, vals, hist, sem0, sem1):
    c = lax.axis_index("c")
    s = lax.axis_index("s")
    wid = c * NS + s

    _zero_rows(vals, CHUNK, D // 16)
    for k in range(ACC_ROWS // (NS * CHUNK)):
        pltpu.sync_copy(vals, hist.at[pl.ds((s * (ACC_ROWS // NS)) + k * CHUNK, CHUNK)])

    lane = lax.iota(jnp.int32, 16)
    onehot0 = jnp.where(lane == 0, 1.0, 0.0).astype(jnp.float32)

    def initrow(i, _):
        vals[i, pl.ds(0, 16)] = onehot0
        return 0

    lax.fori_loop(0, CHUNK, initrow, 0)
    pltpu.sync_copy(packed_hbm.at[wid], packed2)
    plsc.subcore_barrier()

    rbufs = (rb0, rb1)
    sems = (sem0, sem1)
    descs = [None, None]
    for ch in range(NCHUNK):
        p = ch % 2
        if descs[p] is not None:
            descs[p].wait()
        for gi in range(CHUNK // 16):
            r, cc = _unpack(packed2, ch, gi)
            rbufs[p][pl.ds(gi * 16, 16)] = jnp.where(r == cc, TRASH, r)
        descs[p] = pltpu.async_copy(vals, hist.at[rbufs[p]], sems[p], add=True)
    descs[0].wait()
    descs[1].wait()
    plsc.subcore_barrier()

    pltpu.sync_copy(
        hist.at[pl.ds(s * DUMP_ROWS, DUMP_ROWS)],
        out_hbm.at[c, pl.ds(s * DUMP_ROWS, DUMP_ROWS)],
    )


K_FAST = 80
K_SLOW = 80
KMAX = max(K_FAST, K_SLOW)
A_CAP = NS * (K_FAST + K_SLOW) * CHUNK


@functools.partial(
    pl.kernel,
    out_type=jax.ShapeDtypeStruct((NC, ACC_ROWS, D), jnp.float32),
    mesh=_mesh,
    scratch_types=[
        pltpu.VMEM((KMAX, CHUNK), jnp.int32),
        pltpu.VMEM((CHUNK,), jnp.int32),
        pltpu.VMEM((CHUNK,), jnp.int32),
        pltpu.VMEM((CHUNK,), jnp.int32),
        pltpu.VMEM((CHUNK, D), jnp.float32),
        pltpu.VMEM((CHUNK, D), jnp.float32),
        pltpu.VMEM_SHARED((ACC_ROWS, D), jnp.float32),
        pltpu.SemaphoreType.DMA,
        pltpu.SemaphoreType.DMA,
    ],
)
def _agg_call(packed_hbm, g_hbm, out_hbm, packed2, rb0, rb1, cb, rv0, rv1, acc, sem0, sem1):
    c = lax.axis_index("c")
    s = lax.axis_index("s")
    wid = c * NS + s
    my_k = jnp.where(c == 0, K_FAST, K_SLOW)

    _zero_rows(rv0, CHUNK, D // 16)
    for k in range(ACC_ROWS // (NS * CHUNK)):
        pltpu.sync_copy(rv0, acc.at[pl.ds((s * (ACC_ROWS // NS)) + k * CHUNK, CHUNK)])
    pltpu.sync_copy(packed_hbm.at[wid], packed2)
    plsc.subcore_barrier()

    rbufs = (rb0, rb1)
    bufs = (rv0, rv1)
    sems = (sem0, sem1)

    def unpack_rows(ch, dst):
        for gi in range(CHUNK // 16):
            r, _ = _unpack(packed2, ch, gi)
            dst[pl.ds(gi * 16, 16)] = r

    def issue(ch, b):
        unpack_rows(ch, rbufs[b])
        pltpu.async_copy(g_hbm.at[rbufs[b]], bufs[b], sems[b])

    def consume(ch, b):
        for gi in range(CHUNK // 16):
            r, cc = _unpack(packed2, ch, gi)
            cb[pl.ds(gi * 16, 16)] = jnp.where(r == cc, TRASH, cc)
        pltpu.make_async_copy(g_hbm.at[rbufs[b]], bufs[b], sems[b]).wait()
        pltpu.sync_copy(bufs[b], acc.at[cb], add=True)

    issue(0, 0)

    def grp(g2, _):
        for b in range(2):
            ch = g2 * 2 + b

            @pl.when(ch + 1 < my_k)
            def _():
                issue(ch + 1, 1 - b)

            @pl.when(ch < my_k)
            def _():
                consume(ch, b)
        return 0

    lax.fori_loop(0, KMAX // 2, grp, 0)
    plsc.subcore_barrier()

    pltpu.sync_copy(
        acc.at[pl.ds(s * DUMP_ROWS, DUMP_ROWS)],
        out_hbm.at[c, pl.ds(s * DUMP_ROWS, DUMP_ROWS)],
    )


_BR = 1280


def _dis_block(parts_blk):
    deg = 1.0 + jnp.sum(parts_blk, axis=(0, 2))
    return lax.rsqrt(deg)


def _mm_body(x_ref, w_ref, parts_ref, g_ref):
    dis = _dis_block(parts_ref[...])
    h = jnp.dot(x_ref[...], w_ref[...], preferred_element_type=jnp.float32)
    g_ref[...] = h * dis[:, None]


def _fin_body(acc_ref, g_ref, parts_ref, b_ref, o_ref):
    dis = _dis_block(parts_ref[...])
    tot = acc_ref[0] + acc_ref[1] + g_ref[...]
    o_ref[...] = jnp.maximum(tot * dis[:, None] + b_ref[...], 0.0)


def kernel(x, edge_index, W, b):
    ei = edge_index.astype(jnp.int32)
    row, col = ei[0], ei[1]
    pad = E_PAD - row.shape[0]
    if pad:
        zpad = jnp.zeros((pad,), jnp.int32)
        row = jnp.concatenate([row, zpad])
        col = jnp.concatenate([col, zpad])
    packed = jnp.bitwise_or(row, col << 16)
    packed_d = packed.reshape(NW, NCHUNK, CHUNK)

    praw = packed.reshape(-1)[:N_EDGES]
    pa = jnp.concatenate(
        [praw, jnp.zeros((A_CAP - N_EDGES,), jnp.int32)]
    )
    e0 = pa[: NS * K_FAST * CHUNK].reshape(NS, K_FAST, CHUNK)
    e1 = pa[NS * K_FAST * CHUNK :].reshape(NS, K_SLOW, CHUNK)
    e0 = jnp.pad(e0, ((0, 0), (0, KMAX - K_FAST), (0, 0)))
    e1 = jnp.pad(e1, ((0, 0), (0, KMAX - K_SLOW), (0, 0)))
    packed_a = jnp.concatenate([e0[None], e1[None]]).reshape(NW, KMAX, CHUNK)

    parts = _deg_call(packed_d)

    xp = jnp.concatenate(
        [x, jnp.zeros((ACC_ROWS - N_NODES, D), jnp.float32)]
    )

    g = pl.pallas_call(
        _mm_body,
        grid=(ACC_ROWS // _BR,),
        in_specs=[
            pl.BlockSpec((_BR, D), lambda i: (i, 0)),
            pl.BlockSpec((D, D), lambda i: (0, 0)),
            pl.BlockSpec((NC, _BR, D), lambda i: (0, i, 0)),
        ],
        out_specs=pl.BlockSpec((_BR, D), lambda i: (i, 0)),
        out_shape=jax.ShapeDtypeStruct((ACC_ROWS, D), jnp.float32),
    )(xp, W, parts)

    acc = _agg_call(packed_a, g)

    out = pl.pallas_call(
        _fin_body,
        grid=(ACC_ROWS // _BR,),
        in_specs=[
            pl.BlockSpec((NC, _BR, D), lambda i: (0, i, 0)),
            pl.BlockSpec((_BR, D), lambda i: (i, 0)),
            pl.BlockSpec((NC, _BR, D), lambda i: (0, i, 0)),
            pl.BlockSpec((1, D), lambda i: (0, 0)),
        ],
        out_specs=pl.BlockSpec((_BR, D), lambda i: (i, 0)),
        out_shape=jax.ShapeDtypeStruct((ACC_ROWS, D), jnp.float32),
    )(acc, g, parts, b.reshape(1, D))

    return out[:N_NODES]


# speedup vs baseline: 1.7700x; 1.7700x over previous
import functools

import jax
import jax.numpy as jnp
from jax import lax
from jax.experimental import pallas as pl
from jax.experimental.pallas import tpu as pltpu
from jax.experimental.pallas import tpu_sc as plsc

N_NODES = 10000
D = 128
N_EDGES = 320000

NC = 2
NS = 16
NW = NC * NS
CHUNK = 128
EPT = ((N_EDGES + NW * CHUNK - 1) // (NW * CHUNK)) * CHUNK
E_PAD = EPT * NW
NCHUNK = EPT // CHUNK
ACC_ROWS = 10240
TRASH = N_NODES
DUMP_ROWS = ACC_ROWS // NS

_mesh = plsc.VectorSubcoreMesh(core_axis_name="c", subcore_axis_name="s")


def _zero_rows(ref, nrows, ncolgrp):
    z = jnp.zeros((16,), jnp.float32)

    def rowfn(i, _):
        for j in range(ncolgrp):
            ref[i, pl.ds(j * 16, 16)] = z
        return 0

    lax.fori_loop(0, nrows, rowfn, 0)


def _unpack_lo(packed2, ch, gi):
    p = packed2[ch, pl.ds(gi * 16, 16)]
    return jnp.bitwise_and(p, 0xFFFF)


def _unpack_hi(packed2, ch, gi):
    p = packed2[ch, pl.ds(gi * 16, 16)]
    return lax.shift_right_logical(p, 16)


@functools.partial(
    pl.kernel,
    out_type=jax.ShapeDtypeStruct((NC, ACC_ROWS, D), jnp.float32),
    mesh=_mesh,
    scratch_types=[
        pltpu.VMEM((NCHUNK, CHUNK), jnp.int32),
        pltpu.VMEM((CHUNK,), jnp.int32),
        pltpu.VMEM((CHUNK,), jnp.int32),
        pltpu.VMEM((CHUNK, D), jnp.float32),
        pltpu.VMEM_SHARED((ACC_ROWS, D), jnp.float32),
        pltpu.SemaphoreType.DMA,
        pltpu.SemaphoreType.DMA,
    ],
)
def _deg_call(packed_hbm, out_hbm, packed2, rb0, rb1, vals, hist, sem0, sem1):
    c = lax.axis_index("c")
    s = lax.axis_index("s")
    wid = c * NS + s

    _zero_rows(vals, CHUNK, D // 16)
    for k in range(ACC_ROWS // (NS * CHUNK)):
        pltpu.sync_copy(vals, hist.at[pl.ds((s * (ACC_ROWS // NS)) + k * CHUNK, CHUNK)])

    lane = lax.iota(jnp.int32, 16)
    onehot0 = jnp.where(lane == 0, 1.0, 0.0).astype(jnp.float32)

    def initrow(i, _):
        vals[i, pl.ds(0, 16)] = onehot0
        return 0

    lax.fori_loop(0, CHUNK, initrow, 0)
    pltpu.sync_copy(packed_hbm.at[wid], packed2)
    plsc.subcore_barrier()

    rbufs = (rb0, rb1)
    sems = (sem0, sem1)
    descs = [None, None]
    for ch in range(NCHUNK):
        p = ch % 2
        if descs[p] is not None:
            descs[p].wait()
        for gi in range(CHUNK // 16):
            rbufs[p][pl.ds(gi * 16, 16)] = _unpack_lo(packed2, ch, gi)
        descs[p] = pltpu.async_copy(vals, hist.at[rbufs[p]], sems[p], add=True)
    descs[0].wait()
    descs[1].wait()
    plsc.subcore_barrier()

    pltpu.sync_copy(
        hist.at[pl.ds(s * DUMP_ROWS, DUMP_ROWS)],
        out_hbm.at[c, pl.ds(s * DUMP_ROWS, DUMP_ROWS)],
    )


K_FAST = 120
K_SLOW = 38
KMAX = max(K_FAST, K_SLOW)
A_CAP = NS * (K_FAST + K_SLOW) * CHUNK


@functools.partial(
    pl.kernel,
    out_type=jax.ShapeDtypeStruct((NC, ACC_ROWS, D), jnp.float32),
    mesh=_mesh,
    scratch_types=[
        pltpu.VMEM((KMAX, CHUNK), jnp.int32),
        pltpu.VMEM((CHUNK,), jnp.int32),
        pltpu.VMEM((CHUNK,), jnp.int32),
        pltpu.VMEM((CHUNK,), jnp.int32),
        pltpu.VMEM((CHUNK, D), jnp.float32),
        pltpu.VMEM((CHUNK, D), jnp.float32),
        pltpu.VMEM_SHARED((ACC_ROWS, D), jnp.float32),
        pltpu.SemaphoreType.DMA,
        pltpu.SemaphoreType.DMA,
    ],
)
def _agg_call(packed_hbm, g_hbm, out_hbm, packed2, rb0, rb1, cb, rv0, rv1, acc, sem0, sem1):
    c = lax.axis_index("c")
    s = lax.axis_index("s")
    wid = c * NS + s
    my_k = jnp.where(c == 0, K_FAST, K_SLOW)

    _zero_rows(rv0, CHUNK, D // 16)
    for k in range(ACC_ROWS // (NS * CHUNK)):
        pltpu.sync_copy(rv0, acc.at[pl.ds((s * (ACC_ROWS // NS)) + k * CHUNK, CHUNK)])
    pltpu.sync_copy(packed_hbm.at[wid], packed2)
    plsc.subcore_barrier()

    rbufs = (rb0, rb1)
    bufs = (rv0, rv1)
    sems = (sem0, sem1)

    def issue(ch, b):
        for gi in range(CHUNK // 16):
            rbufs[b][pl.ds(gi * 16, 16)] = _unpack_lo(packed2, ch, gi)
        pltpu.async_copy(g_hbm.at[rbufs[b]], bufs[b], sems[b])

    def consume(ch, b):
        for gi in range(CHUNK // 16):
            cb[pl.ds(gi * 16, 16)] = _unpack_hi(packed2, ch, gi)
        pltpu.make_async_copy(g_hbm.at[rbufs[b]], bufs[b], sems[b]).wait()
        pltpu.sync_copy(bufs[b], acc.at[cb], add=True)

    issue(0, 0)
    for ch in range(KMAX):
        p = ch % 2
        if ch + 1 < KMAX:

            @pl.when(ch + 1 < my_k)
            def _():
                issue(ch + 1, 1 - p)

        @pl.when(ch < my_k)
        def _():
            consume(ch, p)

    plsc.subcore_barrier()

    pltpu.sync_copy(
        acc.at[pl.ds(s * DUMP_ROWS, DUMP_ROWS)],
        out_hbm.at[c, pl.ds(s * DUMP_ROWS, DUMP_ROWS)],
    )


_BR = 1280


def _dis_block(parts_blk):
    deg = 1.0 + jnp.sum(parts_blk, axis=(0, 2))
    return lax.rsqrt(deg)


def _mm_body(x_ref, w_ref, parts_ref, g_ref):
    dis = _dis_block(parts_ref[...])
    h = jnp.dot(x_ref[...], w_ref[...], preferred_element_type=jnp.float32)
    g_ref[...] = h * dis[:, None]


def _fin_body(acc_ref, g_ref, parts_ref, b_ref, o_ref):
    dis = _dis_block(parts_ref[...])
    tot = acc_ref[0] + acc_ref[1] + g_ref[...]
    o_ref[...] = jnp.maximum(tot * dis[:, None] + b_ref[...], 0.0)


def kernel(x, edge_index, W, b):
    ei = edge_index.astype(jnp.int32)
    row, col = ei[0], ei[1]
    pad = E_PAD - row.shape[0]
    if pad:
        zpad = jnp.zeros((pad,), jnp.int32)
        row = jnp.concatenate([row, zpad])
        col = jnp.concatenate([col, zpad])
    self_m = row == col
    rr = jnp.where(self_m, TRASH, row)
    rc = jnp.where(self_m, TRASH, col)
    packed = jnp.bitwise_or(rr, rc << 16)
    packed_d = packed.reshape(NW, NCHUNK, CHUNK)

    praw = packed.reshape(-1)[:N_EDGES]
    trash_edge = jnp.int32(TRASH | (TRASH << 16))
    pa = jnp.concatenate(
        [praw, jnp.full((A_CAP - N_EDGES,), trash_edge, jnp.int32)]
    )
    e0 = pa[: NS * K_FAST * CHUNK].reshape(NS, K_FAST, CHUNK)
    e1 = pa[NS * K_FAST * CHUNK :].reshape(NS, K_SLOW, CHUNK)
    e0 = jnp.pad(e0, ((0, 0), (0, KMAX - K_FAST), (0, 0)))
    e1 = jnp.pad(e1, ((0, 0), (0, KMAX - K_SLOW), (0, 0)))
    packed_a = jnp.concatenate([e0[None], e1[None]]).reshape(NW, KMAX, CHUNK)

    parts = _deg_call(packed_d)

    xp = jnp.concatenate(
        [x, jnp.zeros((ACC_ROWS - N_NODES, D), jnp.float32)]
    )

    g = pl.pallas_call(
        _mm_body,
        grid=(ACC_ROWS // _BR,),
        in_specs=[
            pl.BlockSpec((_BR, D), lambda i: (i, 0)),
            pl.BlockSpec((D, D), lambda i: (0, 0)),
            pl.BlockSpec((NC, _BR, D), lambda i: (0, i, 0)),
        ],
        out_specs=pl.BlockSpec((_BR, D), lambda i: (i, 0)),
        out_shape=jax.ShapeDtypeStruct((ACC_ROWS, D), jnp.float32),
    )(xp, W, parts)

    acc = _agg_call(packed_a, g)

    out = pl.pallas_call(
        _fin_body,
        grid=(ACC_ROWS // _BR,),
        in_specs=[
            pl.BlockSpec((NC, _BR, D), lambda i: (0, i, 0)),
            pl.BlockSpec((_BR, D), lambda i: (i, 0)),
            pl.BlockSpec((NC, _BR, D), lambda i: (0, i, 0)),
            pl.BlockSpec((1, D), lambda i: (0, 0)),
        ],
        out_specs=pl.BlockSpec((_BR, D), lambda i: (i, 0)),
        out_shape=jax.ShapeDtypeStruct((ACC_ROWS, D), jnp.float32),
    )(acc, g, parts, b.reshape(1, D))

    return out[:N_NODES]
